# baseline (speedup 1.0000x reference)
import functools

import jax
import jax.numpy as jnp
from jax import lax
from jax.experimental import pallas as pl
from jax.experimental.pallas import tpu as pltpu
from jax.experimental.pallas import tpu_sc as plsc

EPS = 1e-12

_NC = 2
_NS = 16
_NW = _NC * _NS
_CHUNK = 64
_NSLICES = 4
_BT = 512


def _sc_gather(ids_flat, table):
    T = ids_flat.shape[0]
    H = table.shape[1]
    per_w = T // _NW
    n_chunks = per_w // _CHUNK
    mesh = plsc.VectorSubcoreMesh(core_axis_name="c", subcore_axis_name="s")

    @functools.partial(
        pl.kernel,
        out_type=jax.ShapeDtypeStruct((T, H), jnp.float32),
        mesh=mesh,
        scratch_types=[
            pltpu.VMEM((2, _CHUNK), jnp.int32),
            pltpu.VMEM((2, _CHUNK, H), jnp.float32),
            pltpu.SemaphoreType.DMA,
            pltpu.SemaphoreType.DMA,
            pltpu.SemaphoreType.DMA,
            pltpu.SemaphoreType.DMA,
        ],
    )
    def k(ids_hbm, tab_hbm, out_hbm, idx_v, rows_v, g0, g1, s0, s1):
        wid = lax.axis_index("s") * _NC + lax.axis_index("c")
        base = wid * per_w
        gsems = (g0, g1)
        ssems = (s0, s1)
        gcp = [None, None]
        scp = [None, None]
        for i in range(n_chunks):
            b = i % 2
            if scp[b] is not None:
                scp[b].wait()
            pltpu.sync_copy(ids_hbm.at[pl.ds(base + i * _CHUNK, _CHUNK)], idx_v.at[b])
            gcp[b] = pltpu.async_copy(tab_hbm.at[idx_v.at[b]], rows_v.at[b], gsems[b])
            if i > 0:
                pb = 1 - b
                gcp[pb].wait()
                scp[pb] = pltpu.async_copy(
                    rows_v.at[pb],
                    out_hbm.at[pl.ds(base + (i - 1) * _CHUNK, _CHUNK)],
                    ssems[pb],
                )
        last = n_chunks - 1
        lb = last % 2
        gcp[lb].wait()
        pltpu.sync_copy(rows_v.at[lb], out_hbm.at[pl.ds(base + last * _CHUNK, _CHUNK)])
        if scp[1 - lb] is not None:
            scp[1 - lb].wait()

    return k(ids_flat, table)


def _tc_body(g_ref, tt_ref, ent_ref, pos_ref, te_ref, ee_ref, ga_ref, be_ref, *rest):
    o_ref = rest[-1]
    comb8 = jnp.concatenate(
        [te_ref[0, :][None, :] + ee_ref[...], te_ref[1, :][None, :] + ee_ref[...]],
        axis=0,
    )
    idx8 = tt_ref[...] * 4 + ent_ref[...]
    onehot = (idx8 == lax.broadcasted_iota(jnp.int32, (1, 8), 1)).astype(jnp.float32)
    x = g_ref[...] + pos_ref[...] + jnp.dot(
        onehot, comb8, preferred_element_type=jnp.float32
    )
    mean = jnp.mean(x, axis=-1, keepdims=True)
    d = x - mean
    var = jnp.mean(d * d, axis=-1, keepdims=True)
    o_ref[...] = d * lax.rsqrt(var + EPS) * ga_ref[...] + be_ref[...]


def _tc_ln_slice(slice_idx, t_full, gath_s, tt_s, ent_s, pos_emb, type_emb,
                 entity_emb, gamma2, beta2, buf):
    Ts, H = gath_s.shape
    S = pos_emb.shape[0]
    n_s = S // _BT
    n_b = Ts // S
    blk_off = slice_idx * (Ts // _BT)

    in_specs = [
        pl.BlockSpec((_BT, H), lambda s, b: (b * n_s + s, 0)),
        pl.BlockSpec((_BT, 1), lambda s, b: (b * n_s + s, 0)),
        pl.BlockSpec((_BT, 1), lambda s, b: (b * n_s + s, 0)),
        pl.BlockSpec((_BT, H), lambda s, b: (s, 0)),
        pl.BlockSpec((2, H), lambda s, b: (0, 0)),
        pl.BlockSpec((4, H), lambda s, b: (0, 0)),
        pl.BlockSpec((1, H), lambda s, b: (0, 0)),
        pl.BlockSpec((1, H), lambda s, b: (0, 0)),
    ]
    args = [gath_s, tt_s, ent_s, pos_emb, type_emb, entity_emb, gamma2, beta2]
    kwargs = {}
    if buf is not None:
        in_specs.append(pl.BlockSpec(memory_space=pl.ANY))
        args.append(buf)
        kwargs["input_output_aliases"] = {8: 0}

    return pl.pallas_call(
        _tc_body,
        grid=(n_s, n_b),
        in_specs=in_specs,
        out_specs=pl.BlockSpec((_BT, H), lambda s, b: (blk_off + b * n_s + s, 0)),
        out_shape=jax.ShapeDtypeStruct((t_full, H), jnp.float32),
        **kwargs,
    )(*args)


def kernel(input_ids, entity_ids, token_type_ids, word_emb, pos_emb, type_emb, entity_emb, gamma, beta):
    B, S = input_ids.shape
    H = word_emb.shape[1]
    T = B * S
    Ts = T // _NSLICES
    ids = input_ids.reshape(T).astype(jnp.int32)
    tt2 = token_type_ids.reshape(T, 1).astype(jnp.int32)
    ent2 = entity_ids.reshape(T, 1).astype(jnp.int32)
    gamma2 = gamma.reshape(1, H)
    beta2 = beta.reshape(1, H)

    gaths = [
        _sc_gather(lax.slice_in_dim(ids, i * Ts, (i + 1) * Ts), word_emb)
        for i in range(_NSLICES)
    ]
    buf = None
    for i in range(_NSLICES):
        buf = _tc_ln_slice(
            i, T, gaths[i],
            lax.slice_in_dim(tt2, i * Ts, (i + 1) * Ts),
            lax.slice_in_dim(ent2, i * Ts, (i + 1) * Ts),
            pos_emb, type_emb, entity_emb, gamma2, beta2, buf,
        )
    return buf.reshape(B, S, H)


# speedup vs baseline: 1.0500x; 1.0500x over previous
import functools

import jax
import jax.numpy as jnp
from jax import lax
from jax.experimental import pallas as pl
from jax.experimental.pallas import tpu as pltpu
from jax.experimental.pallas import tpu_sc as plsc

EPS = 1e-12

_NC = 2
_NS = 16
_NW = _NC * _NS
_CHUNK = 64
_BT = 512


def _sc_gather(ids_flat, table):
    T = ids_flat.shape[0]
    H = table.shape[1]
    per_w = T // _NW
    n_chunks = per_w // _CHUNK
    mesh = plsc.VectorSubcoreMesh(core_axis_name="c", subcore_axis_name="s")

    @functools.partial(
        pl.kernel,
        out_type=jax.ShapeDtypeStruct((T, H), jnp.float32),
        mesh=mesh,
        scratch_types=[
            pltpu.VMEM((2, _CHUNK), jnp.int32),
            pltpu.VMEM((2, _CHUNK, H), jnp.float32),
            pltpu.SemaphoreType.DMA,
            pltpu.SemaphoreType.DMA,
            pltpu.SemaphoreType.DMA,
            pltpu.SemaphoreType.DMA,
        ],
    )
    def k(ids_hbm, tab_hbm, out_hbm, idx_v, rows_v, g0, g1, s0, s1):
        wid = lax.axis_index("s") * _NC + lax.axis_index("c")
        base = wid * per_w
        gsems = (g0, g1)
        ssems = (s0, s1)
        gcp = [None, None]
        scp = [None, None]
        for i in range(n_chunks):
            b = i % 2
            if scp[b] is not None:
                scp[b].wait()
            pltpu.sync_copy(ids_hbm.at[pl.ds(base + i * _CHUNK, _CHUNK)], idx_v.at[b])
            gcp[b] = pltpu.async_copy(tab_hbm.at[idx_v.at[b]], rows_v.at[b], gsems[b])
            if i > 0:
                pb = 1 - b
                gcp[pb].wait()
                scp[pb] = pltpu.async_copy(
                    rows_v.at[pb],
                    out_hbm.at[pl.ds(base + (i - 1) * _CHUNK, _CHUNK)],
                    ssems[pb],
                )
        last = n_chunks - 1
        lb = last % 2
        gcp[lb].wait()
        pltpu.sync_copy(rows_v.at[lb], out_hbm.at[pl.ds(base + last * _CHUNK, _CHUNK)])
        if scp[1 - lb] is not None:
            scp[1 - lb].wait()

    return k(ids_flat, table)


def _tc_ln(gath, tt2, ent2, pos_emb, type_emb, entity_emb, gamma2, beta2):
    T, H = gath.shape
    S = pos_emb.shape[0]
    n_s = S // _BT
    n_b = T // S
    inv_h = 1.0 / H

    def body(g_ref, tt_ref, ent_ref, pos_ref, te_ref, ee_ref, ga_ref, be_ref, o_ref):
        comb8 = jnp.concatenate(
            [te_ref[0, :][None, :] + ee_ref[...], te_ref[1, :][None, :] + ee_ref[...]],
            axis=0,
        )
        idx8 = tt_ref[...] * 4 + ent_ref[...]
        onehot = (idx8 == lax.broadcasted_iota(jnp.int32, (1, 8), 1)).astype(jnp.float32)
        x = g_ref[...] + pos_ref[...] + jnp.dot(
            onehot, comb8, preferred_element_type=jnp.float32
        )
        ones = jnp.ones((H, 1), dtype=jnp.float32)
        mean = jnp.dot(x, ones, preferred_element_type=jnp.float32) * inv_h
        m2 = jnp.dot(x * x, ones, preferred_element_type=jnp.float32) * inv_h
        var = m2 - mean * mean
        rstd = lax.rsqrt(var + EPS)
        scale = rstd * ga_ref[...]
        shift = be_ref[...] - mean * scale
        o_ref[...] = x * scale + shift

    return pl.pallas_call(
        body,
        grid=(n_s, n_b),
        in_specs=[
            pl.BlockSpec((_BT, H), lambda s, b: (b * n_s + s, 0)),
            pl.BlockSpec((_BT, 1), lambda s, b: (b * n_s + s, 0)),
            pl.BlockSpec((_BT, 1), lambda s, b: (b * n_s + s, 0)),
            pl.BlockSpec((_BT, H), lambda s, b: (s, 0)),
            pl.BlockSpec((2, H), lambda s, b: (0, 0)),
            pl.BlockSpec((4, H), lambda s, b: (0, 0)),
            pl.BlockSpec((1, H), lambda s, b: (0, 0)),
            pl.BlockSpec((1, H), lambda s, b: (0, 0)),
        ],
        out_specs=pl.BlockSpec((_BT, H), lambda s, b: (b * n_s + s, 0)),
        out_shape=jax.ShapeDtypeStruct((T, H), jnp.float32),
    )(gath, tt2, ent2, pos_emb, type_emb, entity_emb, gamma2, beta2)


def kernel(input_ids, entity_ids, token_type_ids, word_emb, pos_emb, type_emb, entity_emb, gamma, beta):
    B, S = input_ids.shape
    H = word_emb.shape[1]
    T = B * S
    ids = input_ids.reshape(T).astype(jnp.int32)
    gath = _sc_gather(ids, word_emb)
    tt2 = token_type_ids.reshape(T, 1).astype(jnp.int32)
    ent2 = entity_ids.reshape(T, 1).astype(jnp.int32)
    out = _tc_ln(
        gath, tt2, ent2, pos_emb, type_emb, entity_emb,
        gamma.reshape(1, H), beta.reshape(1, H),
    )
    return out.reshape(B, S, H)


# speedup vs baseline: 1.1685x; 1.1128x over previous
import functools

import jax
import jax.numpy as jnp
from jax import lax
from jax.experimental import pallas as pl
from jax.experimental.pallas import tpu as pltpu
from jax.experimental.pallas import tpu_sc as plsc

EPS = 1e-12

_NC = 2
_NS = 16
_NW = _NC * _NS
_CHUNK = 64
_BT = 1024


def _sc_gather(ids_flat, table):
    T = ids_flat.shape[0]
    H = table.shape[1]
    per_w = T // _NW
    n_chunks = per_w // _CHUNK
    mesh = plsc.VectorSubcoreMesh(core_axis_name="c", subcore_axis_name="s")

    @functools.partial(
        pl.kernel,
        out_type=jax.ShapeDtypeStruct((T, H), jnp.float32),
        mesh=mesh,
        scratch_types=[
            pltpu.VMEM((per_w,), jnp.int32),
            pltpu.VMEM((2, _CHUNK, H), jnp.float32),
            pltpu.SemaphoreType.DMA,
            pltpu.SemaphoreType.DMA,
            pltpu.SemaphoreType.DMA,
            pltpu.SemaphoreType.DMA,
        ],
    )
    def k(ids_hbm, tab_hbm, out_hbm, idx_v, rows_v, g0, g1, s0, s1):
        wid = lax.axis_index("s") * _NC + lax.axis_index("c")
        base = wid * per_w
        gsems = (g0, g1)
        ssems = (s0, s1)
        gcp = [None, None]
        scp = [None, None]
        pltpu.sync_copy(ids_hbm.at[pl.ds(base, per_w)], idx_v)
        for i in range(n_chunks):
            b = i % 2
            if scp[b] is not None:
                scp[b].wait()
            gcp[b] = pltpu.async_copy(
                tab_hbm.at[idx_v.at[pl.ds(i * _CHUNK, _CHUNK)]], rows_v.at[b], gsems[b]
            )
            if i > 0:
                pb = 1 - b
                gcp[pb].wait()
                scp[pb] = pltpu.async_copy(
                    rows_v.at[pb],
                    out_hbm.at[pl.ds(base + (i - 1) * _CHUNK, _CHUNK)],
                    ssems[pb],
                )
        last = n_chunks - 1
        lb = last % 2
        gcp[lb].wait()
        pltpu.sync_copy(rows_v.at[lb], out_hbm.at[pl.ds(base + last * _CHUNK, _CHUNK)])
        if scp[1 - lb] is not None:
            scp[1 - lb].wait()

    return k(ids_flat, table)


def _tc_ln(gath, tt2, ent2, pos_emb, type_emb, entity_emb, gamma2, beta2):
    T, H = gath.shape
    S = pos_emb.shape[0]
    n_s = S // _BT
    n_b = T // S
    inv_h = 1.0 / H

    def body(g_ref, tt_ref, ent_ref, pos_ref, te_ref, ee_ref, ga_ref, be_ref, o_ref):
        comb8 = jnp.concatenate(
            [te_ref[0, :][None, :] + ee_ref[...], te_ref[1, :][None, :] + ee_ref[...]],
            axis=0,
        )
        idx8 = tt_ref[...] * 4 + ent_ref[...]
        onehot = (idx8 == lax.broadcasted_iota(jnp.int32, (1, 8), 1)).astype(jnp.float32)
        x = g_ref[...] + pos_ref[...] + jnp.dot(
            onehot, comb8, preferred_element_type=jnp.float32
        )
        ones = jnp.ones((H, 1), dtype=jnp.float32)
        mean = jnp.dot(x, ones, preferred_element_type=jnp.float32) * inv_h
        m2 = jnp.dot(x * x, ones, preferred_element_type=jnp.float32) * inv_h
        var = m2 - mean * mean
        rstd = lax.rsqrt(var + EPS)
        scale = rstd * ga_ref[...]
        shift = be_ref[...] - mean * scale
        o_ref[...] = x * scale + shift

    return pl.pallas_call(
        body,
        grid=(n_s, n_b),
        in_specs=[
            pl.BlockSpec((_BT, H), lambda s, b: (b * n_s + s, 0)),
            pl.BlockSpec((_BT, 1), lambda s, b: (b * n_s + s, 0)),
            pl.BlockSpec((_BT, 1), lambda s, b: (b * n_s + s, 0)),
            pl.BlockSpec((_BT, H), lambda s, b: (s, 0)),
            pl.BlockSpec((2, H), lambda s, b: (0, 0)),
            pl.BlockSpec((4, H), lambda s, b: (0, 0)),
            pl.BlockSpec((1, H), lambda s, b: (0, 0)),
            pl.BlockSpec((1, H), lambda s, b: (0, 0)),
        ],
        out_specs=pl.BlockSpec((_BT, H), lambda s, b: (b * n_s + s, 0)),
        out_shape=jax.ShapeDtypeStruct((T, H), jnp.float32),
    )(gath, tt2, ent2, pos_emb, type_emb, entity_emb, gamma2, beta2)


def kernel(input_ids, entity_ids, token_type_ids, word_emb, pos_emb, type_emb, entity_emb, gamma, beta):
    B, S = input_ids.shape
    H = word_emb.shape[1]
    T = B * S
    ids = input_ids.reshape(T).astype(jnp.int32)
    gath = _sc_gather(ids, word_emb)
    tt2 = token_type_ids.reshape(T, 1).astype(jnp.int32)
    ent2 = entity_ids.reshape(T, 1).astype(jnp.int32)
    out = _tc_ln(
        gath, tt2, ent2, pos_emb, type_emb, entity_emb,
        gamma.reshape(1, H), beta.reshape(1, H),
    )
    return out.reshape(B, S, H)


# speedup vs baseline: 1.2119x; 1.0371x over previous
import functools

import jax
import jax.numpy as jnp
from jax import lax
from jax.experimental import pallas as pl
from jax.experimental.pallas import tpu as pltpu
from jax.experimental.pallas import tpu_sc as plsc

EPS = 1e-12

_NC = 2
_NS = 16
_NW = _NC * _NS
_CHUNK = 32
_NBUF = 4
_BT = 2048


def _sc_gather(ids_flat, table):
    T = ids_flat.shape[0]
    H = table.shape[1]
    per_w = T // _NW
    n_chunks = per_w // _CHUNK
    mesh = plsc.VectorSubcoreMesh(core_axis_name="c", subcore_axis_name="s")

    @functools.partial(
        pl.kernel,
        out_type=jax.ShapeDtypeStruct((T, H), jnp.float32),
        mesh=mesh,
        scratch_types=[
            pltpu.VMEM((per_w,), jnp.int32),
            pltpu.VMEM((_NBUF, _CHUNK, H), jnp.float32),
        ]
        + [pltpu.SemaphoreType.DMA] * (2 * _NBUF),
    )
    def k(ids_hbm, tab_hbm, out_hbm, idx_v, rows_v, *sems):
        wid = lax.axis_index("s") * _NC + lax.axis_index("c")
        base = wid * per_w
        gsems = sems[:_NBUF]
        ssems = sems[_NBUF:]
        gcp = [None] * _NBUF
        scp = [None] * _NBUF
        pltpu.sync_copy(ids_hbm.at[pl.ds(base, per_w)], idx_v)
        for i in range(n_chunks):
            b = i % _NBUF
            if scp[b] is not None:
                scp[b].wait()
            gcp[b] = pltpu.async_copy(
                tab_hbm.at[idx_v.at[pl.ds(i * _CHUNK, _CHUNK)]], rows_v.at[b], gsems[b]
            )
            if i >= 2:
                pb = (i - 2) % _NBUF
                gcp[pb].wait()
                scp[pb] = pltpu.async_copy(
                    rows_v.at[pb],
                    out_hbm.at[pl.ds(base + (i - 2) * _CHUNK, _CHUNK)],
                    ssems[pb],
                )
        for j in (n_chunks - 2, n_chunks - 1):
            jb = j % _NBUF
            gcp[jb].wait()
            scp[jb] = pltpu.async_copy(
                rows_v.at[jb], out_hbm.at[pl.ds(base + j * _CHUNK, _CHUNK)], ssems[jb]
            )
        for j in range(_NBUF):
            if scp[j] is not None:
                scp[j].wait()

    return k(ids_flat, table)


def _tc_ln(gath, tt2, ent2, pos_emb, type_emb, entity_emb, gamma2, beta2):
    T, H = gath.shape
    S = pos_emb.shape[0]
    n_s = S // _BT
    n_b = T // S
    inv_h = 1.0 / H

    def body(g_ref, tt_ref, ent_ref, pos_ref, te_ref, ee_ref, ga_ref, be_ref, o_ref):
        comb8 = jnp.concatenate(
            [te_ref[0, :][None, :] + ee_ref[...], te_ref[1, :][None, :] + ee_ref[...]],
            axis=0,
        )
        idx8 = tt_ref[...] * 4 + ent_ref[...]
        onehot = (idx8 == lax.broadcasted_iota(jnp.int32, (1, 8), 1)).astype(jnp.float32)
        x = g_ref[...] + pos_ref[...] + jnp.dot(
            onehot, comb8, preferred_element_type=jnp.float32
        )
        ones = jnp.ones((H, 1), dtype=jnp.float32)
        mean = jnp.dot(x, ones, preferred_element_type=jnp.float32) * inv_h
        m2 = jnp.dot(x * x, ones, preferred_element_type=jnp.float32) * inv_h
        var = m2 - mean * mean
        rstd = lax.rsqrt(var + EPS)
        scale = rstd * ga_ref[...]
        shift = be_ref[...] - mean * scale
        o_ref[...] = x * scale + shift

    return pl.pallas_call(
        body,
        grid=(n_s, n_b),
        in_specs=[
            pl.BlockSpec((_BT, H), lambda s, b: (b * n_s + s, 0)),
            pl.BlockSpec((_BT, 1), lambda s, b: (b * n_s + s, 0)),
            pl.BlockSpec((_BT, 1), lambda s, b: (b * n_s + s, 0)),
            pl.BlockSpec((_BT, H), lambda s, b: (s, 0)),
            pl.BlockSpec((2, H), lambda s, b: (0, 0)),
            pl.BlockSpec((4, H), lambda s, b: (0, 0)),
            pl.BlockSpec((1, H), lambda s, b: (0, 0)),
            pl.BlockSpec((1, H), lambda s, b: (0, 0)),
        ],
        out_specs=pl.BlockSpec((_BT, H), lambda s, b: (b * n_s + s, 0)),
        out_shape=jax.ShapeDtypeStruct((T, H), jnp.float32),
        input_output_aliases={0: 0},
    )(gath, tt2, ent2, pos_emb, type_emb, entity_emb, gamma2, beta2)


def kernel(input_ids, entity_ids, token_type_ids, word_emb, pos_emb, type_emb, entity_emb, gamma, beta):
    B, S = input_ids.shape
    H = word_emb.shape[1]
    T = B * S
    ids = input_ids.reshape(T).astype(jnp.int32)
    gath = _sc_gather(ids, word_emb)
    tt2 = token_type_ids.reshape(T, 1).astype(jnp.int32)
    ent2 = entity_ids.reshape(T, 1).astype(jnp.int32)
    out = _tc_ln(
        gath, tt2, ent2, pos_emb, type_emb, entity_emb,
        gamma.reshape(1, H), beta.reshape(1, H),
    )
    return out.reshape(B, S, H)
